# Initial kernel scaffold; baseline (speedup 1.0000x reference)
import jax
import jax.numpy as jnp
from jax.experimental import pallas as pl


def kernel(atomic_numbers, neighbour_indices, neighbour_displacements, emb_table, td_Wa, td_Wb, mp_Wq, mp_Wk, mp_Wv, mp_Wbasis, mp_ln_scale, mp_ln_bias, out_W1, out_b1, out_ln_scale, out_ln_bias, out_W2, out_b2):
    raise NotImplementedError("write your pallas kernel here")


# speedup vs baseline: 16.6788x; 16.6788x over previous
import functools

import jax
import jax.numpy as jnp
import numpy as np
from jax import lax
from jax.experimental import pallas as pl
from jax.experimental.pallas import tpu as pltpu
from jax.experimental.pallas import tpu_sc as plsc

N_ATOMS = 10000
N_EDGES = 160000
R = 32
F = 32
NB = 16
L2 = 9
CUTOFF = 5.0
D288 = L2 * F

NC = 2
NS = 16
NW = NC * NS
NPW = -(-N_ATOMS // NW)

CHUNK = 64
STRIDE = 56

_SL = [pl.ds(h * 16, 16) for h in range(D288 // 16)]


def _zero_vec():
    return jnp.zeros((16,), jnp.float32)


def _sgather(ref, idx):
    return plsc.load_gather(ref, [jnp.full((16,), idx, jnp.int32)])


def _set_idx1(idx1, val):
    lane0 = lax.iota(jnp.int32, 16) == 0
    plsc.store_scatter(idx1, [jnp.zeros((16,), jnp.int32)],
                       jnp.full((16,), val, jnp.int32), mask=lane0)


def _write_row(src, idx1, out_hbm, row, sem):
    _set_idx1(idx1, row)
    cp = pltpu.make_async_copy(src, out_hbm.at[idx1], sem)
    cp.start()
    cp.wait()


def _zero_rows(zblk, idxZ, out_hbm, n0, n1, sem):
    for r16 in range(16):
        for h in range(D288 // 16):
            zblk[r16, _SL[h]] = _zero_vec()
    nblk = (n1 - n0 + 15) // 16

    def zb(b, _):
        ids = jnp.minimum(n0 + b * 16 + lax.iota(jnp.int32, 16), n1 - 1)
        idxZ[...] = ids
        cp = pltpu.make_async_copy(zblk, out_hbm.at[idxZ], sem)
        cp.start()
        cp.wait()
        return 0

    lax.fori_loop(0, nblk, zb, 0)


def _pass1_body(ge_hbm, perm_hbm, si_hbm, sj_hbm, starts_hbm, an_hbm, emb_hbm,
                y2b_hbm, featp_hbm,
                starts_v, an_v, emb_v, permbuf, ibuf, jbuf, gebuf, featbuf,
                accA, zblk, idx1, idxZ, sem):
    wid = lax.axis_index("c") * NS + lax.axis_index("s")
    pltpu.sync_copy(starts_hbm, starts_v)
    pltpu.sync_copy(an_hbm, an_v)
    pltpu.sync_copy(emb_hbm, emb_v)
    sv = _sgather(starts_v, wid)
    sv1 = _sgather(starts_v, wid + 1)
    e0 = sv[0]
    e1 = sv1[0]
    n0 = wid * NPW
    n1 = jnp.minimum(N_ATOMS, n0 + NPW)

    _zero_rows(zblk, idxZ, y2b_hbm, n0, n1, sem)

    for h in range(D288 // 16):
        accA[0, _SL[h]] = _zero_vec()

    nchunks = (e1 - e0 + STRIDE - 1) // STRIDE

    def chunk_body(c, cur):
        ce = e0 + c * STRIDE
        wce = jnp.minimum(ce, N_EDGES - CHUNK)
        wce = (wce // 8) * 8
        pltpu.sync_copy(perm_hbm.at[pl.ds(wce, CHUNK)], permbuf)
        cps = [
            pltpu.make_async_copy(si_hbm.at[pl.ds(wce, CHUNK)], ibuf, sem),
            pltpu.make_async_copy(sj_hbm.at[pl.ds(wce, CHUNK)], jbuf, sem),
            pltpu.make_async_copy(ge_hbm.at[permbuf], gebuf, sem),
        ]
        for cp in cps:
            cp.start()
        for cp in cps:
            cp.wait()

        def fb(t, _):
            featbuf[t, pl.ds(0, 16)] = gebuf[t, pl.ds(0, 16)]
            featbuf[t, pl.ds(16, 16)] = gebuf[t, pl.ds(48, 16)]
            return 0

        lax.fori_loop(0, CHUNK, fb, 0)
        pltpu.sync_copy(featbuf, featp_hbm.at[pl.ds(wce, CHUNK)])

        lo = ce - wce
        hi = jnp.minimum(ce + STRIDE, e1) - wce

        def edge_body(t, cur):
            i = _sgather(ibuf, t)[0]

            def close(cur):
                _write_row(accA, idx1, y2b_hbm, cur, sem)
                for h in range(D288 // 16):
                    accA[0, _SL[h]] = _zero_vec()
                return i

            def keep(cur):
                return cur

            cur = lax.cond(jnp.logical_and(i != cur, cur >= 0), close, keep, cur)
            cur = jnp.maximum(cur, i)

            j = _sgather(jbuf, t)[0]
            zj = _sgather(an_v, j)[0]
            idx0 = zj * R + lax.iota(jnp.int32, 16)
            er0 = plsc.load_gather(emb_v, [idx0])
            er1 = plsc.load_gather(emb_v, [idx0 + 16])
            rad0 = er0 * gebuf[t, pl.ds(16, 16)]
            rad1 = er1 * gebuf[t, pl.ds(32, 16)]
            g0 = gebuf[t, pl.ds(0, 16)]
            for l in range(L2):
                yl = g0[l]
                accA[0, _SL[2 * l]] = accA[0, _SL[2 * l]] + yl * rad0
                accA[0, _SL[2 * l + 1]] = accA[0, _SL[2 * l + 1]] + yl * rad1
            return cur

        return lax.fori_loop(lo, hi, edge_body, cur)

    cur = lax.fori_loop(0, nchunks, chunk_body, jnp.int32(-1))

    def final_close(cur):
        _write_row(accA, idx1, y2b_hbm, cur, sem)
        return 0

    lax.cond(cur >= 0, final_close, lambda c: 0, cur)


def _pass1_call(ge, perm, si, sj, starts, an, emb_flat):
    mesh = plsc.VectorSubcoreMesh(core_axis_name="c", subcore_axis_name="s",
                                  num_cores=NC, num_subcores=NS)
    f = pl.kernel(
        _pass1_body,
        out_type=[
            jax.ShapeDtypeStruct((N_ATOMS, D288), jnp.float32),
            jax.ShapeDtypeStruct((N_EDGES, 32), jnp.float32),
        ],
        mesh=mesh,
        scratch_types=[
            pltpu.VMEM((64,), jnp.int32),
            pltpu.VMEM((N_ATOMS,), jnp.int32),
            pltpu.VMEM((119 * R,), jnp.float32),
            pltpu.VMEM((CHUNK,), jnp.int32),
            pltpu.VMEM((CHUNK,), jnp.int32),
            pltpu.VMEM((CHUNK,), jnp.int32),
            pltpu.VMEM((CHUNK, 64), jnp.float32),
            pltpu.VMEM((CHUNK, 32), jnp.float32),
            pltpu.VMEM((1, D288), jnp.float32),
            pltpu.VMEM((16, D288), jnp.float32),
            pltpu.VMEM((1,), jnp.int32),
            pltpu.VMEM((16,), jnp.int32),
            pltpu.SemaphoreType.DMA,
        ],
        compiler_params=pltpu.CompilerParams(needs_layout_passes=False, use_tc_tiling_on_sc=False),
    )
    return f(ge, perm, si, sj, starts, an, emb_flat)


def _attn_body(q_hbm, k_hbm, v_hbm, featp_hbm, ws_hbm, si_hbm, sj_hbm,
               starts_hbm, agg_hbm,
               starts_v, ibuf, jbuf, featbuf, wsbuf, qrows, krows, vrows,
               accA, zblk, idx1, idxZ, semA, semB):
    wid = lax.axis_index("c") * NS + lax.axis_index("s")
    pltpu.sync_copy(starts_hbm, starts_v)
    e0 = _sgather(starts_v, wid)[0]
    e1 = _sgather(starts_v, wid + 1)[0]
    n0 = wid * NPW
    n1 = jnp.minimum(N_ATOMS, n0 + NPW)

    _zero_rows(zblk, idxZ, agg_hbm, n0, n1, semA)

    for h in range(D288 // 16):
        accA[0, _SL[h]] = _zero_vec()

    scale = 1.0 / float(np.sqrt(L2 * F))
    nchunks = (e1 - e0 + STRIDE - 1) // STRIDE

    def chunk_body(c, carry):
        cur, m16, d16 = carry
        ce = e0 + c * STRIDE
        wce = jnp.minimum(ce, N_EDGES - CHUNK)
        wce = (wce // 8) * 8
        cps = [
            pltpu.make_async_copy(si_hbm.at[pl.ds(wce, CHUNK)], ibuf, semA),
            pltpu.make_async_copy(sj_hbm.at[pl.ds(wce, CHUNK)], jbuf, semA),
            pltpu.make_async_copy(featp_hbm.at[pl.ds(wce, CHUNK)], featbuf, semB),
            pltpu.make_async_copy(ws_hbm.at[pl.ds(wce, CHUNK)], wsbuf, semB),
        ]
        for cp in cps:
            cp.start()
        cps[0].wait()
        cps[1].wait()
        gps = [
            pltpu.make_async_copy(q_hbm.at[ibuf], qrows, semB),
            pltpu.make_async_copy(k_hbm.at[jbuf], krows, semB),
            pltpu.make_async_copy(v_hbm.at[jbuf], vrows, semB),
        ]
        for cp in gps:
            cp.start()
        cps[2].wait()
        cps[3].wait()
        for cp in gps:
            cp.wait()

        lo = ce - wce
        hi = jnp.minimum(ce + STRIDE, e1) - wce

        def edge_body(t, carry):
            cur, m16, d16 = carry
            i = _sgather(ibuf, t)[0]

            def close(args):
                cur, m16, d16 = args
                inv16 = 1.0 / (d16 + 1e-9)
                for h in range(D288 // 16):
                    accA[0, _SL[h]] = accA[0, _SL[h]] * inv16
                _write_row(accA, idx1, agg_hbm, cur, semA)
                for h in range(D288 // 16):
                    accA[0, _SL[h]] = _zero_vec()
                return i, jnp.full((16,), -1e30, jnp.float32), jnp.zeros((16,), jnp.float32)

            def keep(args):
                return args

            cur, m16, d16 = lax.cond(jnp.logical_and(i != cur, cur >= 0),
                                     close, keep, (cur, m16, d16))
            cur = jnp.maximum(cur, i)

            acc = _zero_vec()
            for h in range(D288 // 16):
                acc = acc + qrows[t, _SL[h]] * krows[t, _SL[h]]
            s = jnp.sum(acc, axis=0) * scale
            s16 = jnp.full((16,), s, jnp.float32)
            frow = featbuf[t, pl.ds(0, 16)]
            cut = frow[9]
            mnew = jnp.maximum(m16, s16)
            rA = jnp.exp(m16 - mnew)
            es = jnp.exp(s16 - mnew) * cut
            d16 = d16 * rA + es
            for l in range(L2):
                yl = frow[l]
                w0 = wsbuf[t, pl.ds(0, 16)] * (es * yl)
                w1 = wsbuf[t, pl.ds(16, 16)] * (es * yl)
                accA[0, _SL[2 * l]] = accA[0, _SL[2 * l]] * rA + w0 * vrows[t, _SL[2 * l]]
                accA[0, _SL[2 * l + 1]] = accA[0, _SL[2 * l + 1]] * rA + w1 * vrows[t, _SL[2 * l + 1]]
            return cur, mnew, d16

        return lax.fori_loop(lo, hi, edge_body, (cur, m16, d16))

    init = (jnp.int32(-1), jnp.full((16,), -1e30, jnp.float32),
            jnp.zeros((16,), jnp.float32))
    cur, m16, d16 = lax.fori_loop(0, nchunks, chunk_body, init)

    def final_close(args):
        cur, d16 = args
        inv16 = 1.0 / (d16 + 1e-9)
        for h in range(D288 // 16):
            accA[0, _SL[h]] = accA[0, _SL[h]] * inv16
        _write_row(accA, idx1, agg_hbm, cur, semA)
        return 0

    lax.cond(cur >= 0, final_close, lambda a: 0, (cur, d16))


def _attn_call(q, k, v, featp, ws, si, sj, starts):
    mesh = plsc.VectorSubcoreMesh(core_axis_name="c", subcore_axis_name="s",
                                  num_cores=NC, num_subcores=NS)
    f = pl.kernel(
        _attn_body,
        out_type=jax.ShapeDtypeStruct((N_ATOMS, D288), jnp.float32),
        mesh=mesh,
        scratch_types=[
            pltpu.VMEM((64,), jnp.int32),
            pltpu.VMEM((CHUNK,), jnp.int32),
            pltpu.VMEM((CHUNK,), jnp.int32),
            pltpu.VMEM((CHUNK, 32), jnp.float32),
            pltpu.VMEM((CHUNK, 32), jnp.float32),
            pltpu.VMEM((CHUNK, D288), jnp.float32),
            pltpu.VMEM((CHUNK, D288), jnp.float32),
            pltpu.VMEM((CHUNK, D288), jnp.float32),
            pltpu.VMEM((1, D288), jnp.float32),
            pltpu.VMEM((16, D288), jnp.float32),
            pltpu.VMEM((1,), jnp.int32),
            pltpu.VMEM((16,), jnp.int32),
            pltpu.SemaphoreType.DMA,
            pltpu.SemaphoreType.DMA,
        ],
        compiler_params=pltpu.CompilerParams(needs_layout_passes=False, use_tc_tiling_on_sc=False),
    )
    return f(q, k, v, featp, ws, si, sj, starts)


BE = 2000
BN = 1000


def _geom_body(d_ref, out_ref):
    d = d_ref[...]
    x0 = d[:, 0:1]
    y0 = d[:, 1:2]
    z0 = d[:, 2:3]
    rr = x0 * x0 + y0 * y0 + z0 * z0
    r = jnp.sqrt(rr)
    inv = 1.0 / (r + 1e-9)
    x = x0 * inv
    y = y0 * inv
    z = z0 * inv
    out_ref[:, 0:1] = jnp.ones_like(x)
    out_ref[:, 1:2] = x
    out_ref[:, 2:3] = y
    out_ref[:, 3:4] = z
    out_ref[:, 4:5] = x * y
    out_ref[:, 5:6] = y * z
    out_ref[:, 6:7] = 0.5 * (3.0 * z * z - 1.0)
    out_ref[:, 7:8] = x * z
    out_ref[:, 8:9] = 0.5 * (x * x - y * y)
    t = jnp.clip(r / CUTOFF, 0.0, 1.0 - 1e-6)
    f = jnp.exp(1.0 - 1.0 / (1.0 - t * t))
    cut = jnp.where(r < CUTOFF, f, 0.0)
    out_ref[:, 9:10] = cut
    out_ref[:, 10:16] = jnp.zeros((d.shape[0], 6), jnp.float32)
    centers = (lax.broadcasted_iota(jnp.int32, (1, R), 1).astype(jnp.float32)
               * (CUTOFF / (R - 1)))
    g = jnp.exp(-(((r - centers) * (R / CUTOFF)) ** 2))
    out_ref[:, 16:48] = g * cut
    centers_b = (lax.broadcasted_iota(jnp.int32, (1, NB), 1).astype(jnp.float32)
                 * (CUTOFF / (NB - 1)))
    gb = jnp.exp(-(((r - centers_b) * (NB / CUTOFF)) ** 2))
    out_ref[:, 48:64] = gb


def _geom_call(d):
    return pl.pallas_call(
        _geom_body,
        grid=(N_EDGES // BE,),
        in_specs=[pl.BlockSpec((BE, 3), lambda i: (i, 0))],
        out_specs=pl.BlockSpec((BE, 64), lambda i: (i, 0)),
        out_shape=jax.ShapeDtypeStruct((N_EDGES, 64), jnp.float32),
    )(d)


def _wbasis_body(featp_ref, wb_ref, *outs):
    g16 = featp_ref[:, 16:32]
    for s in range(6):
        outs[s][...] = jnp.dot(g16, wb_ref[s],
                               preferred_element_type=jnp.float32)


def _wbasis_call(featp, wball):
    return pl.pallas_call(
        _wbasis_body,
        grid=(N_EDGES // BE,),
        in_specs=[pl.BlockSpec((BE, 32), lambda i: (i, 0)),
                  pl.BlockSpec((6, NB, F), lambda i: (0, 0, 0))],
        out_specs=[pl.BlockSpec((BE, F), lambda i: (i, 0))] * 6,
        out_shape=[jax.ShapeDtypeStruct((N_EDGES, F), jnp.float32)] * 6,
    )(featp, wball)


def _td_body(y_ref, wa_ref, wb_ref, out_ref):
    yv = y_ref[...]
    a = jnp.dot(yv, wa_ref[...], preferred_element_type=jnp.float32)
    b = jnp.dot(yv, wb_ref[...], preferred_element_type=jnp.float32)
    a0 = a[:, 0:F]
    b0 = b[:, 0:F]
    for l in range(L2):
        lo, hi = l * F, (l + 1) * F
        out_ref[:, lo:hi] = b[:, lo:hi] * a0 + a[:, lo:hi] * b0


def _td_call(y, wak, wbk):
    return pl.pallas_call(
        _td_body,
        grid=(N_ATOMS // BN,),
        in_specs=[pl.BlockSpec((BN, D288), lambda i: (i, 0)),
                  pl.BlockSpec((D288, D288), lambda i: (0, 0)),
                  pl.BlockSpec((D288, D288), lambda i: (0, 0))],
        out_specs=pl.BlockSpec((BN, D288), lambda i: (i, 0)),
        out_shape=jax.ShapeDtypeStruct((N_ATOMS, D288), jnp.float32),
    )(y, wak, wbk)


def _qkv_body(y_ref, wq_ref, wk_ref, wv_ref, q_ref, k_ref, v_ref):
    yv = y_ref[...]
    q_ref[...] = jnp.dot(yv, wq_ref[...], preferred_element_type=jnp.float32)
    k_ref[...] = jnp.dot(yv, wk_ref[...], preferred_element_type=jnp.float32)
    v_ref[...] = jnp.dot(yv, wv_ref[...], preferred_element_type=jnp.float32)


def _qkv_call(y, wqk, wkk, wvk):
    return pl.pallas_call(
        _qkv_body,
        grid=(N_ATOMS // BN,),
        in_specs=[pl.BlockSpec((BN, D288), lambda i: (i, 0))] +
                 [pl.BlockSpec((D288, D288), lambda i: (0, 0))] * 3,
        out_specs=[pl.BlockSpec((BN, D288), lambda i: (i, 0))] * 3,
        out_shape=[jax.ShapeDtypeStruct((N_ATOMS, D288), jnp.float32)] * 3,
    )(y, wqk, wkk, wvk)


def _ln_block(x, scale, bias):
    outs = []
    for l in range(L2):
        sl = x[:, l * F:(l + 1) * F]
        m = jnp.mean(sl, axis=1, keepdims=True)
        v = jnp.mean((sl - m) * (sl - m), axis=1, keepdims=True)
        outs.append((sl - m) / jnp.sqrt(v + 1e-5))
    return jnp.concatenate(outs, axis=1) * scale + bias


def _lnres_body(y_ref, agg_ref, s_ref, b_ref, out_ref):
    x = y_ref[...] + agg_ref[...]
    out_ref[...] = _ln_block(x, s_ref[...], b_ref[...])


def _lnres_call(y, agg, s, b):
    return pl.pallas_call(
        _lnres_body,
        grid=(N_ATOMS // BN,),
        in_specs=[pl.BlockSpec((BN, D288), lambda i: (i, 0)),
                  pl.BlockSpec((BN, D288), lambda i: (i, 0)),
                  pl.BlockSpec((1, D288), lambda i: (0, 0)),
                  pl.BlockSpec((1, D288), lambda i: (0, 0))],
        out_specs=pl.BlockSpec((BN, D288), lambda i: (i, 0)),
        out_shape=jax.ShapeDtypeStruct((N_ATOMS, D288), jnp.float32),
    )(y, agg, s, b)


def _outhead_body(y_ref, w1_ref, b1_ref, s_ref, b_ref, w2_ref, b2_ref, out_ref):
    yv = y_ref[...]
    y0 = jnp.dot(yv, w1_ref[...], preferred_element_type=jnp.float32) + b1_ref[...]
    h = _ln_block(y0, s_ref[...], b_ref[...])
    h = 0.5 * (jnp.sqrt(h * h + 1.0) - 1.0) + h
    out_ref[...] = (jnp.dot(h, w2_ref[...], preferred_element_type=jnp.float32)
                    + b2_ref[...] + y0)


def _outhead_call(y, w1k, b1, s, b, w2k, b2):
    return pl.pallas_call(
        _outhead_body,
        grid=(N_ATOMS // BN,),
        in_specs=[pl.BlockSpec((BN, D288), lambda i: (i, 0)),
                  pl.BlockSpec((D288, D288), lambda i: (0, 0)),
                  pl.BlockSpec((1, D288), lambda i: (0, 0)),
                  pl.BlockSpec((1, D288), lambda i: (0, 0)),
                  pl.BlockSpec((1, D288), lambda i: (0, 0)),
                  pl.BlockSpec((D288, D288), lambda i: (0, 0)),
                  pl.BlockSpec((1, D288), lambda i: (0, 0))],
        out_specs=pl.BlockSpec((BN, D288), lambda i: (i, 0)),
        out_shape=jax.ShapeDtypeStruct((N_ATOMS, D288), jnp.float32),
    )(y, w1k, b1, s, b, w2k, b2)


def kernel(atomic_numbers, neighbour_indices, neighbour_displacements,
           emb_table, td_Wa, td_Wb, mp_Wq, mp_Wk, mp_Wv, mp_Wbasis,
           mp_ln_scale, mp_ln_bias, out_W1, out_b1, out_ln_scale,
           out_ln_bias, out_W2, out_b2):
    an = atomic_numbers.astype(jnp.int32)
    idx_i = neighbour_indices[:, 0].astype(jnp.int32)
    idx_j = neighbour_indices[:, 1].astype(jnp.int32)
    perm = jnp.argsort(idx_i).astype(jnp.int32)
    si = jnp.take(idx_i, perm)
    sj = jnp.take(idx_j, perm)
    bounds = jnp.arange(NW + 1, dtype=jnp.int32) * NPW
    starts = jnp.searchsorted(si, bounds, side="left").astype(jnp.int32)
    starts = jnp.pad(starts, (0, 64 - (NW + 1)))
    emb_flat = emb_table.astype(jnp.float32).reshape(-1)

    eye9 = np.eye(L2, dtype=np.float32)

    def kron9(w):
        return jnp.kron(jnp.asarray(eye9), w)

    ge = _geom_call(neighbour_displacements.astype(jnp.float32))
    y2b, featp = _pass1_call(ge, perm, si, sj, starts, an, emb_flat)
    ws = _wbasis_call(featp, mp_Wbasis)

    y1 = _td_call(y2b, kron9(td_Wa[0]), kron9(td_Wb[0]))
    y2 = _td_call(y1, kron9(td_Wa[1]), kron9(td_Wb[1]))
    ys = [y2b, y1, y2]

    step = 0
    outs = []
    for bi in range(3):
        yy = ys[bi]
        for _ in range(2):
            q, k, v = _qkv_call(yy, kron9(mp_Wq[step]), kron9(mp_Wk[step]),
                                kron9(mp_Wv[step]))
            agg = _attn_call(q, k, v, featp, ws[step], si, sj, starts)
            yy = _lnres_call(yy, agg,
                             jnp.tile(mp_ln_scale[step], L2)[None, :],
                             jnp.tile(mp_ln_bias[step], L2)[None, :])
            step += 1
        h = _outhead_call(yy, kron9(out_W1[bi]),
                          jnp.tile(out_b1[bi], L2)[None, :],
                          jnp.tile(out_ln_scale[bi], L2)[None, :],
                          jnp.tile(out_ln_bias[bi], L2)[None, :],
                          kron9(out_W2[bi]),
                          jnp.tile(out_b2[bi], L2)[None, :])
        outs.append(h.reshape(N_ATOMS, L2, F))
    return jnp.concatenate(outs, axis=-1)
